# baseline (speedup 1.0000x reference)
import functools

import jax
import jax.numpy as jnp
from jax import lax
from jax.experimental import pallas as pl
from jax.experimental.pallas import tpu as pltpu
from jax.experimental.pallas import tpu_sc as plsc

NNODES = 10000
NPAD = 10240
NC, NS, LANES = 2, 16, 16
NW = NC * NS
EB = 128
IC = 16
RB = 1024

_MESH = plsc.VectorSubcoreMesh(core_axis_name="c", subcore_axis_name="s")


def _degree_kernel(edges_hbm):
    kd = edges_hbm.shape[2]
    rows = NPAD // NS

    @functools.partial(
        pl.kernel,
        out_type=jax.ShapeDtypeStruct((NC, NPAD, LANES), jnp.float32),
        mesh=_MESH,
        compiler_params=pltpu.CompilerParams(use_tc_tiling_on_sc=False),
        scratch_types=[
            pltpu.VMEM((kd, EB), jnp.int32),
            pltpu.VMEM((EB, LANES), jnp.float32),
            pltpu.VMEM_SHARED((NPAD, LANES), jnp.float32),
        ],
    )
    def k(e_hbm, deg_hbm, idx_v, buf_v, acc_sh):
        c = lax.axis_index("c")
        s = lax.axis_index("s")

        @pl.loop(0, EB)
        def _(i):
            buf_v[i, :] = jnp.zeros((LANES,), jnp.float32)

        for t in range(rows // EB):
            pltpu.sync_copy(buf_v, acc_sh.at[pl.ds(s * rows + t * EB, EB)])
        plsc.subcore_barrier()

        @pl.loop(0, EB)
        def _(i):
            buf_v[i, :] = jnp.ones((LANES,), jnp.float32)

        pltpu.sync_copy(e_hbm.at[c, s], idx_v)

        @pl.loop(0, kd)
        def _(j):
            pltpu.sync_copy(buf_v, acc_sh.at[idx_v.at[j]], add=True)

        plsc.subcore_barrier()
        pltpu.sync_copy(acc_sh.at[pl.ds(s * rows, rows)],
                        deg_hbm.at[c, pl.ds(s * rows, rows)])

    return k(edges_hbm)


def _prop_kernel(table_hbm, edges_hbm):
    d = table_hbm.shape[1]
    k_blocks = edges_hbm.shape[1]
    rows = NPAD // NS
    nbuf = 2 if d >= 128 else 4
    nch = k_blocks // IC
    assert k_blocks % IC == 0 and IC % nbuf == 0

    @functools.partial(
        pl.kernel,
        out_type=jax.ShapeDtypeStruct((NC, NPAD, d), jnp.float32),
        mesh=_MESH,
        compiler_params=pltpu.CompilerParams(use_tc_tiling_on_sc=False),
        scratch_types=(
            [pltpu.VMEM((IC, 2, EB), jnp.int32)] * 2
            + [pltpu.VMEM((EB, d), jnp.float32)] * nbuf
            + [pltpu.VMEM_SHARED((NPAD, d), jnp.float32)]
            + [pltpu.SemaphoreType.DMA] * (2 + nbuf)
        ),
    )
    def k(tab_hbm, e_hbm, out_hbm, *scratch):
        idx_v = list(scratch[0:2])
        rows_v = list(scratch[2:2 + nbuf])
        acc_sh = scratch[2 + nbuf]
        isem = list(scratch[3 + nbuf:5 + nbuf])
        gsem = list(scratch[5 + nbuf:5 + 2 * nbuf])
        c = lax.axis_index("c")
        s = lax.axis_index("s")
        w = c * NS + s

        @pl.loop(0, EB)
        def _(i):
            @pl.loop(0, d, step=LANES)
            def _(jc):
                rows_v[0][i, pl.ds(jc, LANES)] = jnp.zeros((LANES,), jnp.float32)

        for t in range(rows // EB):
            pltpu.sync_copy(rows_v[0], acc_sh.at[pl.ds(s * rows + t * EB, EB)])
        plsc.subcore_barrier()

        pltpu.async_copy(e_hbm.at[w, pl.ds(0, IC)], idx_v[0], isem[0])
        for ch in range(nch):
            p = ch % 2
            pltpu.make_async_copy(
                e_hbm.at[w, pl.ds(ch * IC, IC)], idx_v[p], isem[p]).wait()
            if ch + 1 < nch:
                pltpu.async_copy(
                    e_hbm.at[w, pl.ds((ch + 1) * IC, IC)],
                    idx_v[1 - p], isem[1 - p])
            for b in range(nbuf):
                pltpu.async_copy(
                    tab_hbm.at[idx_v[p].at[b, 0]], rows_v[b], gsem[b])

            @pl.loop(0, IC, step=nbuf)
            def _(j):
                for b in range(nbuf):
                    pltpu.make_async_copy(
                        tab_hbm.at[idx_v[p].at[j + b, 0]],
                        rows_v[b], gsem[b]).wait()
                    pltpu.sync_copy(
                        rows_v[b], acc_sh.at[idx_v[p].at[j + b, 1]], add=True)

                    @pl.when(j + nbuf + b < IC)
                    def _():
                        pltpu.async_copy(
                            tab_hbm.at[idx_v[p].at[j + nbuf + b, 0]],
                            rows_v[b], gsem[b])

        plsc.subcore_barrier()
        pltpu.sync_copy(acc_sh.at[pl.ds(s * rows, rows)],
                        out_hbm.at[c, pl.ds(s * rows, rows)])

    return k(table_hbm, edges_hbm)


def _scale_x(xp, deg_rep):
    nfeat = xp.shape[1]

    def body(x_ref, d_ref, o_ref):
        ns = lax.rsqrt(jnp.maximum(d_ref[0, :, 0:1], 1.0))
        o_ref[...] = x_ref[...] * ns

    return pl.pallas_call(
        body,
        grid=(NPAD // RB,),
        in_specs=[
            pl.BlockSpec((RB, nfeat), lambda i: (i, 0)),
            pl.BlockSpec((NC, RB, LANES), lambda i: (0, i, 0)),
        ],
        out_specs=pl.BlockSpec((RB, nfeat), lambda i: (i, 0)),
        out_shape=jax.ShapeDtypeStruct((NPAD, nfeat), jnp.float32),
    )(xp, deg_rep)


def _fused_mlp(agg1, deg_rep, W1, b1_2d, W2):
    nfeat, nhid = W1.shape
    nclass = W2.shape[1]

    def body(a_ref, d_ref, w1_ref, b1_ref, w2_ref, o_ref):
        a = a_ref[0] + a_ref[1]
        nd = lax.rsqrt(jnp.maximum(d_ref[1, :, 0:1], 1.0))
        ns = lax.rsqrt(jnp.maximum(d_ref[0, :, 0:1], 1.0))
        h = jnp.dot(a * nd, w1_ref[...], preferred_element_type=jnp.float32)
        h = jnp.maximum(h + b1_ref[...], 0.0)
        y = jnp.dot(h, w2_ref[...], preferred_element_type=jnp.float32)
        o_ref[...] = y * ns

    return pl.pallas_call(
        body,
        grid=(NPAD // RB,),
        in_specs=[
            pl.BlockSpec((NC, RB, nfeat), lambda i: (0, i, 0)),
            pl.BlockSpec((NC, RB, LANES), lambda i: (0, i, 0)),
            pl.BlockSpec((nfeat, nhid), lambda i: (0, 0)),
            pl.BlockSpec((1, nhid), lambda i: (0, 0)),
            pl.BlockSpec((nhid, nclass), lambda i: (0, 0)),
        ],
        out_specs=pl.BlockSpec((RB, nclass), lambda i: (i, 0)),
        out_shape=jax.ShapeDtypeStruct((NPAD, nclass), jnp.float32),
    )(agg1, deg_rep, W1, b1_2d, W2)


def _finish(agg2, deg_rep, b2_2d):
    nclass = agg2.shape[2]

    def body(a_ref, d_ref, b_ref, o_ref):
        a = a_ref[0] + a_ref[1]
        nd = lax.rsqrt(jnp.maximum(d_ref[1, :, 0:1], 1.0))
        o_ref[...] = a * nd + b_ref[...]

    return pl.pallas_call(
        body,
        grid=(NPAD // RB,),
        in_specs=[
            pl.BlockSpec((NC, RB, nclass), lambda i: (0, i, 0)),
            pl.BlockSpec((NC, RB, LANES), lambda i: (0, i, 0)),
            pl.BlockSpec((1, nclass), lambda i: (0, 0)),
        ],
        out_specs=pl.BlockSpec((RB, nclass), lambda i: (i, 0)),
        out_shape=jax.ShapeDtypeStruct((NPAD, nclass), jnp.float32),
    )(agg2, deg_rep, b2_2d)


@jax.jit
def _gcn(x, edge_index, W1, b1, W2, b2):
    src = edge_index[0].astype(jnp.int32)
    dst = edge_index[1].astype(jnp.int32)
    n, nfeat = x.shape
    e = src.shape[0]

    k_blocks = -(-e // (NW * EB))
    k_blocks = -(-k_blocks // IC) * IC
    epad = NW * k_blocks * EB
    fill = jnp.full((epad - e,), NNODES, jnp.int32)
    srcp = jnp.concatenate([src, fill])
    dstp = jnp.concatenate([dst, fill])
    edges = jnp.stack(
        [srcp.reshape(NW, k_blocks, EB), dstp.reshape(NW, k_blocks, EB)],
        axis=2)
    kd = epad // (NS * EB)
    edges_deg = jnp.stack(
        [srcp.reshape(NS, kd, EB), dstp.reshape(NS, kd, EB)])

    deg_rep = _degree_kernel(edges_deg)

    xp = jnp.zeros((NPAD, nfeat), jnp.float32).at[:n].set(x)
    xs = _scale_x(xp, deg_rep)
    agg1 = _prop_kernel(xs, edges)
    ys = _fused_mlp(agg1, deg_rep, W1, b1.reshape(1, -1), W2)
    agg2 = _prop_kernel(ys, edges)
    outp = _finish(agg2, deg_rep, b2.reshape(1, -1))
    return outp[:n]


def kernel(x, edge_index, W1, b1, W2, b2):
    return _gcn(x, edge_index, W1, b1, W2, b2)


# speedup vs baseline: 1.4941x; 1.4941x over previous
import functools

import jax
import jax.numpy as jnp
from jax import lax
from jax.experimental import pallas as pl
from jax.experimental.pallas import tpu as pltpu
from jax.experimental.pallas import tpu_sc as plsc

NNODES = 10000
NPAD = 10240
NC, NS, LANES = 2, 16, 16
NW = NC * NS
EB = 128
F0 = 0.364
RB = 1024

_MESH = plsc.VectorSubcoreMesh(core_axis_name="c", subcore_axis_name="s")


def _degree_kernel(edges_hbm):
    kd = edges_hbm.shape[2]
    rows = NPAD // NS

    @functools.partial(
        pl.kernel,
        out_type=jax.ShapeDtypeStruct((NC, NPAD, LANES), jnp.float32),
        mesh=_MESH,
        compiler_params=pltpu.CompilerParams(use_tc_tiling_on_sc=False),
        scratch_types=[
            pltpu.VMEM((kd, EB), jnp.int32),
            pltpu.VMEM((EB, LANES), jnp.float32),
            pltpu.VMEM_SHARED((NPAD, LANES), jnp.float32),
        ],
    )
    def k(e_hbm, deg_hbm, idx_v, buf_v, acc_sh):
        c = lax.axis_index("c")
        s = lax.axis_index("s")

        @pl.loop(0, EB)
        def _(i):
            buf_v[i, :] = jnp.zeros((LANES,), jnp.float32)

        for t in range(rows // EB):
            pltpu.sync_copy(buf_v, acc_sh.at[pl.ds(s * rows + t * EB, EB)])
        plsc.subcore_barrier()

        @pl.loop(0, EB)
        def _(i):
            buf_v[i, :] = jnp.ones((LANES,), jnp.float32)

        pltpu.sync_copy(e_hbm.at[c, s], idx_v)

        @pl.loop(0, kd)
        def _(j):
            pltpu.sync_copy(buf_v, acc_sh.at[idx_v.at[j]], add=True)

        plsc.subcore_barrier()
        pltpu.sync_copy(acc_sh.at[pl.ds(s * rows, rows)],
                        deg_hbm.at[c, pl.ds(s * rows, rows)])

    return k(edges_hbm)


def _prop_kernel(table_hbm, edges_hbm, kb0, kb1):
    d = table_hbm.shape[1]
    k_max = edges_hbm.shape[1]
    rows = NPAD // NS

    @functools.partial(
        pl.kernel,
        out_type=jax.ShapeDtypeStruct((NC, NPAD, d), jnp.float32),
        mesh=_MESH,
        compiler_params=pltpu.CompilerParams(use_tc_tiling_on_sc=False),
        scratch_types=[
            pltpu.VMEM((k_max, 2, EB), jnp.int32),
            pltpu.VMEM((EB, d), jnp.float32),
            pltpu.VMEM_SHARED((NPAD, d), jnp.float32),
            pltpu.SemaphoreType.DMA,
        ],
    )
    def k(tab_hbm, e_hbm, out_hbm, idx_v, rows_v, acc_sh, sem):
        c = lax.axis_index("c")
        s = lax.axis_index("s")
        w = c * NS + s
        kb = jnp.where(c == 0, kb0, kb1)

        @pl.loop(0, EB)
        def _(i):
            @pl.loop(0, d, step=LANES)
            def _(jc):
                rows_v[i, pl.ds(jc, LANES)] = jnp.zeros((LANES,), jnp.float32)

        for t in range(rows // EB):
            pltpu.sync_copy(rows_v, acc_sh.at[pl.ds(s * rows + t * EB, EB)])
        plsc.subcore_barrier()

        pltpu.sync_copy(e_hbm.at[w], idx_v)

        @pl.loop(0, kb)
        def _(j):
            pltpu.async_copy(tab_hbm.at[idx_v.at[j, 0]], rows_v, sem).wait()
            pltpu.sync_copy(rows_v, acc_sh.at[idx_v.at[j, 1]], add=True)

        plsc.subcore_barrier()
        pltpu.sync_copy(acc_sh.at[pl.ds(s * rows, rows)],
                        out_hbm.at[c, pl.ds(s * rows, rows)])

    return k(table_hbm, edges_hbm)


def _scale_x(xp, deg_rep):
    nfeat = xp.shape[1]

    def body(x_ref, d_ref, o_ref):
        ns = lax.rsqrt(jnp.maximum(d_ref[0, :, 0:1], 1.0))
        o_ref[...] = x_ref[...] * ns

    return pl.pallas_call(
        body,
        grid=(NPAD // RB,),
        in_specs=[
            pl.BlockSpec((RB, nfeat), lambda i: (i, 0)),
            pl.BlockSpec((NC, RB, LANES), lambda i: (0, i, 0)),
        ],
        out_specs=pl.BlockSpec((RB, nfeat), lambda i: (i, 0)),
        out_shape=jax.ShapeDtypeStruct((NPAD, nfeat), jnp.float32),
    )(xp, deg_rep)


def _fused_mlp(agg1, deg_rep, W1, b1_2d, W2):
    nfeat, nhid = W1.shape
    nclass = W2.shape[1]

    def body(a_ref, d_ref, w1_ref, b1_ref, w2_ref, o_ref):
        a = a_ref[0] + a_ref[1]
        nd = lax.rsqrt(jnp.maximum(d_ref[1, :, 0:1], 1.0))
        ns = lax.rsqrt(jnp.maximum(d_ref[0, :, 0:1], 1.0))
        h = jnp.dot(a * nd, w1_ref[...], preferred_element_type=jnp.float32)
        h = jnp.maximum(h + b1_ref[...], 0.0)
        y = jnp.dot(h, w2_ref[...], preferred_element_type=jnp.float32)
        o_ref[...] = y * ns

    return pl.pallas_call(
        body,
        grid=(NPAD // RB,),
        in_specs=[
            pl.BlockSpec((NC, RB, nfeat), lambda i: (0, i, 0)),
            pl.BlockSpec((NC, RB, LANES), lambda i: (0, i, 0)),
            pl.BlockSpec((nfeat, nhid), lambda i: (0, 0)),
            pl.BlockSpec((1, nhid), lambda i: (0, 0)),
            pl.BlockSpec((nhid, nclass), lambda i: (0, 0)),
        ],
        out_specs=pl.BlockSpec((RB, nclass), lambda i: (i, 0)),
        out_shape=jax.ShapeDtypeStruct((NPAD, nclass), jnp.float32),
    )(agg1, deg_rep, W1, b1_2d, W2)


def _finish(agg2, deg_rep, b2_2d):
    nclass = agg2.shape[2]

    def body(a_ref, d_ref, b_ref, o_ref):
        a = a_ref[0] + a_ref[1]
        nd = lax.rsqrt(jnp.maximum(d_ref[1, :, 0:1], 1.0))
        o_ref[...] = a * nd + b_ref[...]

    return pl.pallas_call(
        body,
        grid=(NPAD // RB,),
        in_specs=[
            pl.BlockSpec((NC, RB, nclass), lambda i: (0, i, 0)),
            pl.BlockSpec((NC, RB, LANES), lambda i: (0, i, 0)),
            pl.BlockSpec((1, nclass), lambda i: (0, 0)),
        ],
        out_specs=pl.BlockSpec((RB, nclass), lambda i: (i, 0)),
        out_shape=jax.ShapeDtypeStruct((NPAD, nclass), jnp.float32),
    )(agg2, deg_rep, b2_2d)


@jax.jit
def _gcn(x, edge_index, W1, b1, W2, b2):
    src = edge_index[0].astype(jnp.int32)
    dst = edge_index[1].astype(jnp.int32)
    n, nfeat = x.shape
    e = src.shape[0]

    blocks = -(-e // EB)
    kb0 = max(1, min(int(round(blocks * F0 / NS)), blocks // NS))
    b0 = kb0 * NS
    kb1 = -(-(blocks - b0) // NS)
    kmax = max(kb0, kb1)
    epad = (b0 + kb1 * NS) * EB
    fill = jnp.full((epad - e,), NNODES, jnp.int32)
    srcp = jnp.concatenate([src, fill])
    dstp = jnp.concatenate([dst, fill])

    def _slabs(a):
        p0 = a[:b0 * EB].reshape(NS, kb0, EB)
        p1 = a[b0 * EB:].reshape(NS, kb1, EB)
        p0 = jnp.concatenate(
            [p0, jnp.full((NS, kmax - kb0, EB), NNODES, jnp.int32)], axis=1)
        p1 = jnp.concatenate(
            [p1, jnp.full((NS, kmax - kb1, EB), NNODES, jnp.int32)], axis=1)
        return jnp.concatenate([p0, p1], axis=0)

    edges = jnp.stack([_slabs(srcp), _slabs(dstp)], axis=2)
    kd = epad // (NS * EB)
    edges_deg = jnp.stack(
        [srcp.reshape(NS, kd, EB), dstp.reshape(NS, kd, EB)])

    deg_rep = _degree_kernel(edges_deg)

    xp = jnp.zeros((NPAD, nfeat), jnp.float32).at[:n].set(x)
    xs = _scale_x(xp, deg_rep)
    agg1 = _prop_kernel(xs, edges, kb0, kb1)
    ys = _fused_mlp(agg1, deg_rep, W1, b1.reshape(1, -1), W2)
    agg2 = _prop_kernel(ys, edges, kb0, kb1)
    outp = _finish(agg2, deg_rep, b2.reshape(1, -1))
    return outp[:n]


def kernel(x, edge_index, W1, b1, W2, b2):
    return _gcn(x, edge_index, W1, b1, W2, b2)


# speedup vs baseline: 1.8633x; 1.2471x over previous
import functools

import jax
import jax.numpy as jnp
from jax import lax
from jax.experimental import pallas as pl
from jax.experimental.pallas import tpu as pltpu
from jax.experimental.pallas import tpu_sc as plsc

NNODES = 10000
NPAD = 10240
NC, NS, LANES = 2, 16, 16
NW = NC * NS
EB = 128
F0 = 0.555
RB = 1024

_MESH = plsc.VectorSubcoreMesh(core_axis_name="c", subcore_axis_name="s")


def _degree_kernel(edges_hbm):
    kd = edges_hbm.shape[2]
    rows = NPAD // NS

    @functools.partial(
        pl.kernel,
        out_type=jax.ShapeDtypeStruct((NC, NPAD, LANES), jnp.float32),
        mesh=_MESH,
        compiler_params=pltpu.CompilerParams(use_tc_tiling_on_sc=False),
        scratch_types=[
            pltpu.VMEM((kd, EB), jnp.int32),
            pltpu.VMEM((EB, LANES), jnp.float32),
            pltpu.VMEM_SHARED((NPAD, LANES), jnp.float32),
        ],
    )
    def k(e_hbm, deg_hbm, idx_v, buf_v, acc_sh):
        c = lax.axis_index("c")
        s = lax.axis_index("s")

        @pl.loop(0, EB)
        def _(i):
            buf_v[i, :] = jnp.zeros((LANES,), jnp.float32)

        for t in range(rows // EB):
            pltpu.sync_copy(buf_v, acc_sh.at[pl.ds(s * rows + t * EB, EB)])
        plsc.subcore_barrier()

        @pl.loop(0, EB)
        def _(i):
            buf_v[i, :] = jnp.ones((LANES,), jnp.float32)

        pltpu.sync_copy(e_hbm.at[c, s], idx_v)

        @pl.loop(0, kd)
        def _(j):
            pltpu.sync_copy(buf_v, acc_sh.at[idx_v.at[j]], add=True)

        plsc.subcore_barrier()
        pltpu.sync_copy(acc_sh.at[pl.ds(s * rows, rows)],
                        deg_hbm.at[c, pl.ds(s * rows, rows)])

    return k(edges_hbm)


def _prop_kernel(table_hbm, edges_hbm, kb0, kb1):
    d = table_hbm.shape[1]
    k_max = edges_hbm.shape[1]
    rows = NPAD // NS

    @functools.partial(
        pl.kernel,
        out_type=jax.ShapeDtypeStruct((NC, NPAD, d), jnp.float32),
        mesh=_MESH,
        compiler_params=pltpu.CompilerParams(use_tc_tiling_on_sc=False),
        scratch_types=[
            pltpu.VMEM((k_max, 2, EB), jnp.int32),
            pltpu.VMEM((EB, d), jnp.float32),
            pltpu.VMEM_SHARED((NPAD, d), jnp.float32),
            pltpu.SemaphoreType.DMA,
        ],
    )
    def k(tab_hbm, e_hbm, out_hbm, idx_v, rows_v, acc_sh, sem):
        c = lax.axis_index("c")
        s = lax.axis_index("s")
        w = c * NS + s
        kb = jnp.where(c == 0, kb0, kb1)

        @pl.loop(0, EB)
        def _(i):
            @pl.loop(0, d, step=LANES)
            def _(jc):
                rows_v[i, pl.ds(jc, LANES)] = jnp.zeros((LANES,), jnp.float32)

        for t in range(rows // EB):
            pltpu.sync_copy(rows_v, acc_sh.at[pl.ds(s * rows + t * EB, EB)])
        plsc.subcore_barrier()

        pltpu.sync_copy(e_hbm.at[w], idx_v)

        @pl.loop(0, kb)
        def _(j):
            pltpu.async_copy(tab_hbm.at[idx_v.at[j, 0]], rows_v, sem).wait()
            pltpu.sync_copy(rows_v, acc_sh.at[idx_v.at[j, 1]], add=True)

        plsc.subcore_barrier()
        pltpu.sync_copy(acc_sh.at[pl.ds(s * rows, rows)],
                        out_hbm.at[c, pl.ds(s * rows, rows)])

    return k(table_hbm, edges_hbm)


def _scale_x(xp, deg_rep):
    nfeat = xp.shape[1]

    def body(x_ref, d_ref, o_ref):
        ns = lax.rsqrt(jnp.maximum(d_ref[0, :, 0:1], 1.0))
        o_ref[...] = x_ref[...] * ns

    return pl.pallas_call(
        body,
        grid=(NPAD // RB,),
        in_specs=[
            pl.BlockSpec((RB, nfeat), lambda i: (i, 0)),
            pl.BlockSpec((NC, RB, LANES), lambda i: (0, i, 0)),
        ],
        out_specs=pl.BlockSpec((RB, nfeat), lambda i: (i, 0)),
        out_shape=jax.ShapeDtypeStruct((NPAD, nfeat), jnp.float32),
    )(xp, deg_rep)


def _fused_mlp(agg1, deg_rep, W1, b1_2d, W2):
    nfeat, nhid = W1.shape
    nclass = W2.shape[1]

    def body(a_ref, d_ref, w1_ref, b1_ref, w2_ref, o_ref):
        a = a_ref[0] + a_ref[1]
        nd = lax.rsqrt(jnp.maximum(d_ref[1, :, 0:1], 1.0))
        ns = lax.rsqrt(jnp.maximum(d_ref[0, :, 0:1], 1.0))
        h = jnp.dot(a * nd, w1_ref[...], preferred_element_type=jnp.float32)
        h = jnp.maximum(h + b1_ref[...], 0.0)
        y = jnp.dot(h, w2_ref[...], preferred_element_type=jnp.float32)
        o_ref[...] = y * ns

    return pl.pallas_call(
        body,
        grid=(NPAD // RB,),
        in_specs=[
            pl.BlockSpec((NC, RB, nfeat), lambda i: (0, i, 0)),
            pl.BlockSpec((NC, RB, LANES), lambda i: (0, i, 0)),
            pl.BlockSpec((nfeat, nhid), lambda i: (0, 0)),
            pl.BlockSpec((1, nhid), lambda i: (0, 0)),
            pl.BlockSpec((nhid, nclass), lambda i: (0, 0)),
        ],
        out_specs=pl.BlockSpec((RB, nclass), lambda i: (i, 0)),
        out_shape=jax.ShapeDtypeStruct((NPAD, nclass), jnp.float32),
    )(agg1, deg_rep, W1, b1_2d, W2)


def _finish(agg2, deg_rep, b2_2d):
    nclass = agg2.shape[2]

    def body(a_ref, d_ref, b_ref, o_ref):
        a = a_ref[0] + a_ref[1]
        nd = lax.rsqrt(jnp.maximum(d_ref[1, :, 0:1], 1.0))
        o_ref[...] = a * nd + b_ref[...]

    return pl.pallas_call(
        body,
        grid=(NPAD // RB,),
        in_specs=[
            pl.BlockSpec((NC, RB, nclass), lambda i: (0, i, 0)),
            pl.BlockSpec((NC, RB, LANES), lambda i: (0, i, 0)),
            pl.BlockSpec((1, nclass), lambda i: (0, 0)),
        ],
        out_specs=pl.BlockSpec((RB, nclass), lambda i: (i, 0)),
        out_shape=jax.ShapeDtypeStruct((NPAD, nclass), jnp.float32),
    )(agg2, deg_rep, b2_2d)


@jax.jit
def _gcn(x, edge_index, W1, b1, W2, b2):
    src = edge_index[0].astype(jnp.int32)
    dst = edge_index[1].astype(jnp.int32)
    n, nfeat = x.shape
    e = src.shape[0]

    blocks = -(-e // EB)
    kb0 = max(1, min(int(round(blocks * F0 / NS)), blocks // NS))
    b0 = kb0 * NS
    kb1 = -(-(blocks - b0) // NS)
    kmax = max(kb0, kb1)
    epad = (b0 + kb1 * NS) * EB
    fill = jnp.full((epad - e,), NNODES, jnp.int32)
    srcp = jnp.concatenate([src, fill])
    dstp = jnp.concatenate([dst, fill])

    def _slabs(a):
        p0 = a[:b0 * EB].reshape(NS, kb0, EB)
        p1 = a[b0 * EB:].reshape(NS, kb1, EB)
        p0 = jnp.concatenate(
            [p0, jnp.full((NS, kmax - kb0, EB), NNODES, jnp.int32)], axis=1)
        p1 = jnp.concatenate(
            [p1, jnp.full((NS, kmax - kb1, EB), NNODES, jnp.int32)], axis=1)
        return jnp.concatenate([p0, p1], axis=0)

    edges = jnp.stack([_slabs(srcp), _slabs(dstp)], axis=2)
    kd = epad // (NS * EB)
    edges_deg = jnp.stack(
        [srcp.reshape(NS, kd, EB), dstp.reshape(NS, kd, EB)])

    deg_rep = _degree_kernel(edges_deg)

    xp = jnp.zeros((NPAD, nfeat), jnp.float32).at[:n].set(x)
    xs = _scale_x(xp, deg_rep)
    agg1 = _prop_kernel(xs, edges, kb0, kb1)
    ys = _fused_mlp(agg1, deg_rep, W1, b1.reshape(1, -1), W2)
    agg2 = _prop_kernel(ys, edges, kb0, kb1)
    outp = _finish(agg2, deg_rep, b2.reshape(1, -1))
    return outp[:n]


def kernel(x, edge_index, W1, b1, W2, b2):
    return _gcn(x, edge_index, W1, b1, W2, b2)
